# baseline (speedup 1.0000x reference)
import functools

import jax
import jax.numpy as jnp
from jax import lax
from jax.experimental import pallas as pl
from jax.experimental.pallas import tpu as pltpu
from jax.experimental.pallas import tpu_sc as plsc

NSC = 2
NSUB = 16
LK = 112
G = 10


def _sc_mesh():
    return plsc.VectorSubcoreMesh(core_axis_name="c", subcore_axis_name="s")


def _deg_body(NP, CH, dst_hbm, deg_out, dst_l, zbuf, obuf, deg_acc, t0):
    c = lax.axis_index("c")
    s = lax.axis_index("s")
    rpw = NP // NSUB
    zero = jnp.zeros((16,), jnp.float32)
    one = jnp.ones((16,), jnp.float32)

    def _init(i, carry):
        zbuf[pl.ds(i * 16, 16)] = zero
        return carry
    lax.fori_loop(0, rpw // 16, _init, 0)

    def _init1(i, carry):
        obuf[pl.ds(i * 16, 16)] = one
        return carry
    lax.fori_loop(0, LK // 16, _init1, 0)

    pltpu.sync_copy(dst_hbm.at[c, s], dst_l)
    pltpu.sync_copy(zbuf, deg_acc.at[pl.ds(s * rpw, rpw)])
    plsc.subcore_barrier()

    def _scat(j, carry):
        pltpu.async_copy(obuf, deg_acc.at[dst_l.at[j]], t0, add=True)
        return carry
    lax.fori_loop(0, CH, _scat, 0)

    def _drain(j, carry):
        pltpu.make_async_copy(obuf, deg_acc.at[dst_l.at[0]], t0).wait()
        return carry
    lax.fori_loop(0, CH, _drain, 0)
    plsc.subcore_barrier()
    pltpu.sync_copy(deg_acc.at[pl.ds(s * rpw, rpw)],
                    deg_out.at[c, pl.ds(s * rpw, rpw)])


def _make_deg(NP, CH):
    return pl.kernel(
        functools.partial(_deg_body, NP, CH),
        out_type=jax.ShapeDtypeStruct((NSC, NP), jnp.float32),
        mesh=_sc_mesh(),
        scratch_types=[
            pltpu.VMEM((CH, LK), jnp.int32),
            pltpu.VMEM((NP // NSUB,), jnp.float32),
            pltpu.VMEM((LK,), jnp.float32),
            pltpu.VMEM_SHARED((NP,), jnp.float32),
            pltpu.SemaphoreType.DMA,
        ],
    )


def _prop_body(NP, D, NG, u_hbm, sd_hbm, v_out,
               sd0, sd1, rows0, rows1, rows2, acc,
               g0, g1, g2, t0, t1, t2, semi0, semi1, semu):
    c = lax.axis_index("c")
    s = lax.axis_index("s")
    rpw = NP // NSUB
    CH = NG * G
    sd_bufs = (sd0, sd1)
    isems = (semi0, semi1)
    rows = (rows0, rows1, rows2)
    gsems = (g0, g1, g2)
    tsems = (t0, t1, t2)

    pltpu.async_copy(sd_hbm.at[c, s, 0], sd0, semi0)
    if NG > 1:
        pltpu.async_copy(sd_hbm.at[c, s, 1], sd1, semi1)

    @pl.when(c == 0)
    def _():
        pltpu.async_copy(u_hbm.at[pl.ds(s * rpw, rpw)],
                         acc.at[pl.ds(s * rpw, rpw)], semu)
        pltpu.make_async_copy(u_hbm.at[pl.ds(s * rpw, rpw)],
                              acc.at[pl.ds(s * rpw, rpw)], semu).wait()

    @pl.when(c == 1)
    def _():
        zero = jnp.zeros((16,), jnp.float32)

        def _z(j, carry):
            for q in range(D // 16):
                rows0[j, pl.ds(q * 16, 16)] = zero
            return carry
        lax.fori_loop(0, LK, _z, 0)
        off = 0
        while off < rpw:
            n = min(LK, rpw - off)
            pltpu.sync_copy(rows0.at[pl.ds(0, n)],
                            acc.at[pl.ds(s * rpw + off, n)])
            off += n

    plsc.subcore_barrier()

    def _gather(j):
        sl = rows[j % 3]
        pltpu.async_copy(u_hbm.at[sd_bufs[(j // G) % 2].at[j % G]],
                         sl, gsems[j % 3])

    def _wait_gather(j):
        pltpu.make_async_copy(u_hbm.at[sd_bufs[(j // G) % 2].at[j % G]],
                              rows[j % 3], gsems[j % 3]).wait()

    def _scat(j):
        pltpu.async_copy(rows[j % 3],
                         acc.at[sd_bufs[(j // G) % 2].at[G + (j % G)]],
                         tsems[j % 3], add=True)

    def _drain_scat(j):
        pltpu.make_async_copy(rows[j % 3],
                              acc.at[sd_bufs[(j // G) % 2].at[G + (j % G)]],
                              tsems[j % 3]).wait()

    pltpu.make_async_copy(sd_hbm.at[c, s, 0], sd0, semi0).wait()
    _gather(0)
    if CH > 1:
        _gather(1)
    for j in range(CH):
        _wait_gather(j)
        _scat(j)
        if j % G == 1 and 1 <= j // G < NG - 1:
            gn = j // G + 1
            pltpu.async_copy(sd_hbm.at[c, s, gn],
                             sd_bufs[gn % 2], isems[gn % 2])
        jn = j + 2
        if jn < CH:
            if j >= 1:
                _drain_scat(jn)
            if jn % G == 0:
                pltpu.make_async_copy(sd_hbm.at[c, s, jn // G],
                                      sd_bufs[(jn // G) % 2],
                                      isems[(jn // G) % 2]).wait()
            _gather(jn)
    _drain_scat(CH - 3)
    _drain_scat(CH - 2)
    _drain_scat(CH - 1)
    plsc.subcore_barrier()
    pltpu.sync_copy(acc.at[pl.ds(s * rpw, rpw)],
                    v_out.at[c, pl.ds(s * rpw, rpw), :])


def _make_prop(NP, D, NG):
    return pl.kernel(
        functools.partial(_prop_body, NP, D, NG),
        out_type=jax.ShapeDtypeStruct((NSC, NP, D), jnp.float32),
        mesh=_sc_mesh(),
        scratch_types=[
            pltpu.VMEM((2 * G, LK), jnp.int32),
            pltpu.VMEM((2 * G, LK), jnp.int32),
            pltpu.VMEM((LK, D), jnp.float32),
            pltpu.VMEM((LK, D), jnp.float32),
            pltpu.VMEM((LK, D), jnp.float32),
            pltpu.VMEM_SHARED((NP, D), jnp.float32),
            pltpu.SemaphoreType.DMA,
            pltpu.SemaphoreType.DMA,
            pltpu.SemaphoreType.DMA,
            pltpu.SemaphoreType.DMA,
            pltpu.SemaphoreType.DMA,
            pltpu.SemaphoreType.DMA,
            pltpu.SemaphoreType.DMA,
            pltpu.SemaphoreType.DMA,
            pltpu.SemaphoreType.DMA,
        ],
    )


def _scale_body(p_ref, x_ref, o_ref):
    deg = p_ref[0] + p_ref[1] + 1.0
    o_ref[...] = lax.rsqrt(deg) * x_ref[...]


def _combine_body(p_ref, va_ref, vb_ref, o_ref):
    deg = p_ref[0] + p_ref[1] + 1.0
    o_ref[...] = (va_ref[0] + vb_ref[0]) / deg


def _final_body(p_ref, va_ref, vb_ref, w_ref, b_ref, o_ref):
    deg = p_ref[0] + p_ref[1] + 1.0
    h = lax.rsqrt(deg) * (va_ref[0] + vb_ref[0])
    o_ref[...] = jnp.dot(h, w_ref[...],
                         preferred_element_type=jnp.float32) + b_ref[...]


def kernel(x, edge_index, W, b):
    N, D = x.shape
    E = edge_index.shape[1]
    f32 = jnp.float32

    NP = ((N + 1279) // 1280) * 1280
    RB = NP // 10
    CH = G * (-(-E // (NSC * NSUB * LK * G)))
    NG = CH // G
    EP = NSC * NSUB * LK * CH

    pad_n = EP - E
    pad_idx = N + (jnp.arange(pad_n, dtype=jnp.int32) % (NP - N))
    src = jnp.concatenate([edge_index[0], pad_idx]).reshape(NSC, NSUB, CH, LK)
    dst = jnp.concatenate([edge_index[1], pad_idx]).reshape(NSC, NSUB, CH, LK)
    sd = jnp.concatenate(
        [src.reshape(NSC, NSUB, NG, G, LK),
         dst.reshape(NSC, NSUB, NG, G, LK)], axis=3)

    deg_parts = _make_deg(NP, CH)(dst).reshape(NSC, NP, 1)

    nblk = NP // RB
    row_spec = pl.BlockSpec((RB, D), lambda i: (i, 0))
    p_spec = pl.BlockSpec((NSC, RB, 1), lambda i: (0, i, 0))
    va_spec = pl.BlockSpec((1, RB, D), lambda i: (0, i, 0))
    vb_spec = pl.BlockSpec((1, RB, D), lambda i: (1, i, 0))

    u0 = pl.pallas_call(
        _scale_body,
        grid=(nblk,),
        in_specs=[p_spec, row_spec],
        out_specs=row_spec,
        out_shape=jax.ShapeDtypeStruct((NP, D), f32),
    )(deg_parts, x)

    prop = _make_prop(NP, D, NG)
    v1 = prop(u0, sd)

    u1 = pl.pallas_call(
        _combine_body,
        grid=(nblk,),
        in_specs=[p_spec, va_spec, vb_spec],
        out_specs=row_spec,
        out_shape=jax.ShapeDtypeStruct((NP, D), f32),
    )(deg_parts, v1, v1)

    v2 = prop(u1, sd)

    out = pl.pallas_call(
        _final_body,
        grid=(nblk,),
        in_specs=[p_spec, va_spec, vb_spec,
                  pl.BlockSpec((D, D), lambda i: (0, 0)),
                  pl.BlockSpec((1, D), lambda i: (0, 0))],
        out_specs=row_spec,
        out_shape=jax.ShapeDtypeStruct((N, D), f32),
    )(deg_parts, v2, v2, W, b.reshape(1, D))

    return out


# speedup vs baseline: 1.0235x; 1.0235x over previous
import functools

import jax
import jax.numpy as jnp
from jax import lax
from jax.experimental import pallas as pl
from jax.experimental.pallas import tpu as pltpu
from jax.experimental.pallas import tpu_sc as plsc

NSC = 2
NSUB = 16
LK = 112
G = 10


def _sc_mesh():
    return plsc.VectorSubcoreMesh(core_axis_name="c", subcore_axis_name="s")


def _deg_body(NP, CH, dst_hbm, deg_out, dst_l, zbuf, obuf, deg_acc, t0):
    c = lax.axis_index("c")
    s = lax.axis_index("s")
    rpw = NP // NSUB
    zero = jnp.zeros((16,), jnp.float32)
    one = jnp.ones((16,), jnp.float32)

    def _init(i, carry):
        zbuf[pl.ds(i * 16, 16)] = zero
        return carry
    lax.fori_loop(0, rpw // 16, _init, 0)

    def _init1(i, carry):
        obuf[pl.ds(i * 16, 16)] = one
        return carry
    lax.fori_loop(0, LK // 16, _init1, 0)

    pltpu.sync_copy(dst_hbm.at[c, s], dst_l)
    pltpu.sync_copy(zbuf, deg_acc.at[pl.ds(s * rpw, rpw)])
    plsc.subcore_barrier()

    def _scat(j, carry):
        pltpu.async_copy(obuf, deg_acc.at[dst_l.at[j]], t0, add=True)
        return carry
    lax.fori_loop(0, CH, _scat, 0)

    def _drain(j, carry):
        pltpu.make_async_copy(obuf, deg_acc.at[dst_l.at[0]], t0).wait()
        return carry
    lax.fori_loop(0, CH, _drain, 0)
    plsc.subcore_barrier()
    pltpu.sync_copy(deg_acc.at[pl.ds(s * rpw, rpw)],
                    deg_out.at[c, pl.ds(s * rpw, rpw)])


def _make_deg(NP, CH):
    return pl.kernel(
        functools.partial(_deg_body, NP, CH),
        out_type=jax.ShapeDtypeStruct((NSC, NP), jnp.float32),
        mesh=_sc_mesh(),
        scratch_types=[
            pltpu.VMEM((CH, LK), jnp.int32),
            pltpu.VMEM((NP // NSUB,), jnp.float32),
            pltpu.VMEM((LK,), jnp.float32),
            pltpu.VMEM_SHARED((NP,), jnp.float32),
            pltpu.SemaphoreType.DMA,
        ],
    )


def _prop_body(NP, D, NG, u_hbm, sd_hbm, v_out,
               sd0, sd1, rows0, rows1, rows2, acc,
               g0, g1, g2, t0, t1, t2, semi0, semi1, semu):
    c = lax.axis_index("c")
    s = lax.axis_index("s")
    rpw = NP // NSUB
    CH = NG * G
    sd_bufs = (sd0, sd1)
    isems = (semi0, semi1)
    rows = (rows0, rows1, rows2)
    gsems = (g0, g1, g2)
    tsems = (t0, t1, t2)

    pltpu.async_copy(sd_hbm.at[c, s, 0], sd0, semi0)
    if NG > 1:
        pltpu.async_copy(sd_hbm.at[c, s, 1], sd1, semi1)

    @pl.when(c == 0)
    def _():
        pltpu.async_copy(u_hbm.at[pl.ds(s * rpw, rpw)],
                         acc.at[pl.ds(s * rpw, rpw)], semu)
        pltpu.make_async_copy(u_hbm.at[pl.ds(s * rpw, rpw)],
                              acc.at[pl.ds(s * rpw, rpw)], semu).wait()

    @pl.when(c == 1)
    def _():
        zero = jnp.zeros((16,), jnp.float32)

        def _z(j, carry):
            for q in range(D // 16):
                rows0[j, pl.ds(q * 16, 16)] = zero
            return carry
        lax.fori_loop(0, LK, _z, 0)
        off = 0
        while off < rpw:
            n = min(LK, rpw - off)
            pltpu.sync_copy(rows0.at[pl.ds(0, n)],
                            acc.at[pl.ds(s * rpw + off, n)])
            off += n

    plsc.subcore_barrier()

    def _gather(j):
        sl = rows[j % 3]
        pltpu.async_copy(u_hbm.at[sd_bufs[(j // G) % 2].at[j % G]],
                         sl, gsems[j % 3])

    def _wait_gather(j):
        pltpu.make_async_copy(u_hbm.at[sd_bufs[(j // G) % 2].at[j % G]],
                              rows[j % 3], gsems[j % 3]).wait()

    def _scat(j):
        pltpu.async_copy(rows[j % 3],
                         acc.at[sd_bufs[(j // G) % 2].at[G + (j % G)]],
                         tsems[j % 3], add=True)

    def _drain_scat(j):
        pltpu.make_async_copy(rows[j % 3],
                              acc.at[sd_bufs[(j // G) % 2].at[G + (j % G)]],
                              tsems[j % 3]).wait()

    pltpu.make_async_copy(sd_hbm.at[c, s, 0], sd0, semi0).wait()
    _gather(0)
    if CH > 1:
        _gather(1)
    for j in range(CH):
        _wait_gather(j)
        _scat(j)
        if j % G == 1 and 1 <= j // G < NG - 1:
            gn = j // G + 1
            pltpu.async_copy(sd_hbm.at[c, s, gn],
                             sd_bufs[gn % 2], isems[gn % 2])
        jn = j + 2
        if jn < CH:
            if j >= 1:
                _drain_scat(jn)
            if jn % G == 0:
                pltpu.make_async_copy(sd_hbm.at[c, s, jn // G],
                                      sd_bufs[(jn // G) % 2],
                                      isems[(jn // G) % 2]).wait()
            _gather(jn)
    _drain_scat(CH - 3)
    _drain_scat(CH - 2)
    _drain_scat(CH - 1)
    plsc.subcore_barrier()
    pltpu.sync_copy(acc.at[pl.ds(s * rpw, rpw)],
                    v_out.at[c, pl.ds(s * rpw, rpw), :])


def _make_prop(NP, D, NG):
    return pl.kernel(
        functools.partial(_prop_body, NP, D, NG),
        out_type=jax.ShapeDtypeStruct((NSC, NP, D), jnp.float32),
        mesh=_sc_mesh(),
        scratch_types=[
            pltpu.VMEM((2 * G, LK), jnp.int32),
            pltpu.VMEM((2 * G, LK), jnp.int32),
            pltpu.VMEM((LK, D), jnp.float32),
            pltpu.VMEM((LK, D), jnp.float32),
            pltpu.VMEM((LK, D), jnp.float32),
            pltpu.VMEM_SHARED((NP, D), jnp.float32),
            pltpu.SemaphoreType.DMA,
            pltpu.SemaphoreType.DMA,
            pltpu.SemaphoreType.DMA,
            pltpu.SemaphoreType.DMA,
            pltpu.SemaphoreType.DMA,
            pltpu.SemaphoreType.DMA,
            pltpu.SemaphoreType.DMA,
            pltpu.SemaphoreType.DMA,
            pltpu.SemaphoreType.DMA,
        ],
    )


def _scale_body(p_ref, x_ref, o_ref):
    deg = p_ref[0] + p_ref[1] + 1.0
    o_ref[...] = lax.rsqrt(deg) * x_ref[...]


def _combine_body(p_ref, va_ref, vb_ref, o_ref):
    deg = p_ref[0] + p_ref[1] + 1.0
    o_ref[...] = (va_ref[0] + vb_ref[0]) / deg


def _final_body(p_ref, va_ref, vb_ref, w_ref, b_ref, o_ref):
    deg = p_ref[0] + p_ref[1] + 1.0
    h = lax.rsqrt(deg) * (va_ref[0] + vb_ref[0])
    o_ref[...] = jnp.dot(h, w_ref[...],
                         preferred_element_type=jnp.float32) + b_ref[...]


def kernel(x, edge_index, W, b):
    N, D = x.shape
    E = edge_index.shape[1]
    f32 = jnp.float32

    NP = ((N + 1279) // 1280) * 1280
    RB = NP // 5
    CH = G * (-(-E // (NSC * NSUB * LK * G)))
    NG = CH // G
    EP = NSC * NSUB * LK * CH

    pad_n = EP - E
    pad_idx = N + (jnp.arange(pad_n, dtype=jnp.int32) % (NP - N))
    src = jnp.concatenate([edge_index[0], pad_idx]).reshape(NSC, NSUB, CH, LK)
    dst = jnp.concatenate([edge_index[1], pad_idx]).reshape(NSC, NSUB, CH, LK)
    sd = jnp.concatenate(
        [src.reshape(NSC, NSUB, NG, G, LK),
         dst.reshape(NSC, NSUB, NG, G, LK)], axis=3)

    deg_parts = _make_deg(NP, CH)(dst).reshape(NSC, NP, 1)

    nblk = NP // RB
    row_spec = pl.BlockSpec((RB, D), lambda i: (i, 0))
    p_spec = pl.BlockSpec((NSC, RB, 1), lambda i: (0, i, 0))
    va_spec = pl.BlockSpec((1, RB, D), lambda i: (0, i, 0))
    vb_spec = pl.BlockSpec((1, RB, D), lambda i: (1, i, 0))

    u0 = pl.pallas_call(
        _scale_body,
        grid=(nblk,),
        in_specs=[p_spec, row_spec],
        out_specs=row_spec,
        out_shape=jax.ShapeDtypeStruct((NP, D), f32),
    )(deg_parts, x)

    prop = _make_prop(NP, D, NG)
    v1 = prop(u0, sd)

    u1 = pl.pallas_call(
        _combine_body,
        grid=(nblk,),
        in_specs=[p_spec, va_spec, vb_spec],
        out_specs=row_spec,
        out_shape=jax.ShapeDtypeStruct((NP, D), f32),
    )(deg_parts, v1, v1)

    v2 = prop(u1, sd)

    out = pl.pallas_call(
        _final_body,
        grid=(nblk,),
        in_specs=[p_spec, va_spec, vb_spec,
                  pl.BlockSpec((D, D), lambda i: (0, 0)),
                  pl.BlockSpec((1, D), lambda i: (0, 0))],
        out_specs=row_spec,
        out_shape=jax.ShapeDtypeStruct((N, D), f32),
    )(deg_parts, v2, v2, W, b.reshape(1, D))

    return out
